# baseline (speedup 1.0000x reference)
import functools

import numpy as np

import jax
import jax.numpy as jnp
from jax import lax
from jax.experimental import pallas as pl
from jax.experimental.pallas import tpu as pltpu
from jax.experimental.pallas import tpu_sc as plsc

N = 10000
E = 320000
R = 8
B = 16

NC = 2
NS = 16
LANES = 16

CHUNK = 80
SUPER = 2000
G16 = CHUNK // LANES

E_PER_CORE = E // NC
E_PER_TILE_B = E // (NC * NS)
E_PER_TILE_P = E // NS
NR = N * R
ROWS_PER_TILE = N // NS
RT = 624


_GATHER_DNUMS = lax.GatherDimensionNumbers(
    offset_dims=(), collapsed_slice_dims=(0,), start_index_map=(0,))


def _bcast16(vec16, lane):
  idx = jnp.full((LANES, 1), lane, dtype=jnp.int32)
  return lax.gather(vec16, idx, _GATHER_DNUMS, slice_sizes=(1,),
                    mode=lax.GatherScatterMode.PROMISE_IN_BOUNDS)


def _prep_body(src_hbm, dst_hbm, et_hbm, srcrel_hbm, norm_hbm,
               cnt_sh, zbuf, onesv, dstv, etv, srcv, comp80, cv80,
               nrmv, srv, sem):
  cid = lax.axis_index("c")
  sid = lax.axis_index("s")
  wid = cid * NS + sid

  def _z(i, _):
    zbuf[pl.ds(pl.multiple_of(i * LANES, LANES), LANES)] = jnp.zeros(
        (LANES,), jnp.float32)
    return 0
  lax.fori_loop(0, SUPER // LANES, _z, 0)
  for g in range(G16):
    onesv[pl.ds(g * LANES, LANES)] = jnp.ones((LANES,), jnp.float32)

  base_z = sid * (NR // NS)
  pltpu.sync_copy(zbuf, cnt_sh.at[pl.ds(base_z, SUPER)])
  pltpu.sync_copy(zbuf, cnt_sh.at[pl.ds(base_z + SUPER, SUPER)])
  pltpu.sync_copy(zbuf.at[pl.ds(0, NR // NS - 2 * SUPER)],
                  cnt_sh.at[pl.ds(base_z + 2 * SUPER, NR // NS - 2 * SUPER)])
  plsc.subcore_barrier()

  def _count_super(s, _):
    base = sid * E_PER_TILE_P + s * SUPER
    pltpu.sync_copy(dst_hbm.at[pl.ds(base, SUPER)], dstv)
    pltpu.sync_copy(et_hbm.at[pl.ds(base, SUPER)], etv)

    def _count_chunk(j, _):
      off = pl.multiple_of(j * CHUNK, CHUNK)
      for g in range(G16):
        o = pl.multiple_of(off + g * LANES, LANES)
        comp80[pl.ds(g * LANES, LANES)] = (
            dstv[pl.ds(o, LANES)] * R + etv[pl.ds(o, LANES)])
      pltpu.sync_copy(onesv, cnt_sh.at[comp80], add=True)
      return 0
    lax.fori_loop(0, SUPER // CHUNK, _count_chunk, 0)
    return 0
  lax.fori_loop(0, E_PER_TILE_P // SUPER, _count_super, 0)
  plsc.subcore_barrier()

  def _norm_super(s, _):
    base = wid * E_PER_TILE_B + s * SUPER
    pltpu.sync_copy(dst_hbm.at[pl.ds(base, SUPER)], dstv)
    pltpu.sync_copy(et_hbm.at[pl.ds(base, SUPER)], etv)
    pltpu.sync_copy(src_hbm.at[pl.ds(base, SUPER)], srcv)

    def _srv(i, _):
      o = pl.multiple_of(i * LANES, LANES)
      srv[pl.ds(o, LANES)] = etv[pl.ds(o, LANES)] * N + srcv[pl.ds(o, LANES)]
      return 0
    lax.fori_loop(0, SUPER // LANES, _srv, 0)

    def _norm_chunk(j, _):
      off = pl.multiple_of(j * CHUNK, CHUNK)
      for g in range(G16):
        o = pl.multiple_of(off + g * LANES, LANES)
        comp80[pl.ds(g * LANES, LANES)] = (
            dstv[pl.ds(o, LANES)] * R + etv[pl.ds(o, LANES)])
      pltpu.async_copy(cnt_sh.at[comp80], cv80, sem).wait()
      for g in range(G16):
        o = pl.multiple_of(off + g * LANES, LANES)
        nrmv[pl.ds(o, LANES)] = 1.0 / jnp.maximum(
            cv80[pl.ds(g * LANES, LANES)], 1.0)
      return 0
    lax.fori_loop(0, SUPER // CHUNK, _norm_chunk, 0)

    pltpu.sync_copy(nrmv, norm_hbm.at[pl.ds(base, SUPER)])
    pltpu.sync_copy(srv, srcrel_hbm.at[pl.ds(base, SUPER)])
    return 0
  lax.fori_loop(0, E_PER_TILE_B // SUPER, _norm_super, 0)


@functools.partial(
    pl.kernel,
    out_type=(jax.ShapeDtypeStruct((E,), jnp.int32),
              jax.ShapeDtypeStruct((E,), jnp.float32)),
    mesh=plsc.VectorSubcoreMesh(core_axis_name="c", subcore_axis_name="s"),
    scratch_types=(
        pltpu.VMEM_SHARED((NR,), jnp.float32),
        pltpu.VMEM((SUPER,), jnp.float32),
        pltpu.VMEM((CHUNK,), jnp.float32),
        pltpu.VMEM((SUPER,), jnp.int32),
        pltpu.VMEM((SUPER,), jnp.int32),
        pltpu.VMEM((SUPER,), jnp.int32),
        pltpu.VMEM((CHUNK,), jnp.int32),
        pltpu.VMEM((CHUNK,), jnp.float32),
        pltpu.VMEM((SUPER,), jnp.float32),
        pltpu.VMEM((SUPER,), jnp.int32),
        pltpu.SemaphoreType.DMA,
    ),
)
def _sc_prep(src_hbm, dst_hbm, et_hbm, srcrel_hbm, norm_hbm, *rest):
  _prep_body(src_hbm, dst_hbm, et_hbm, srcrel_hbm, norm_hbm, *rest)


def _agg_body(xwr_hbm, srcrel_hbm, dst_hbm, norm_hbm, aggp_hbm,
              agg_sh, rows0, rows1, rowsf, rowsf1, srv, dstv, nv,
              dst80_0, dst80_1, sem0, sem1, ssem0, ssem1):
  cid = lax.axis_index("c")
  sid = lax.axis_index("s")
  wid = cid * NS + sid
  rows_v = rowsf

  def _z(i, _):
    for c in range(8):
      rows_v[i, pl.ds(c * LANES, LANES)] = jnp.zeros((LANES,), jnp.float32)
    return 0
  lax.fori_loop(0, CHUNK, _z, 0)
  rbase = sid * RT
  for j in range(RT // CHUNK):
    pltpu.sync_copy(rows_v, agg_sh.at[pl.ds(rbase + j * CHUNK, CHUNK)])
  pltpu.sync_copy(rows_v.at[pl.ds(0, RT % CHUNK)],
                  agg_sh.at[pl.ds(rbase + (RT // CHUNK) * CHUNK, RT % CHUNK)])

  @pl.when(sid == NS - 1)
  def _zlast():
    pltpu.sync_copy(rows_v.at[pl.ds(0, N - NS * RT)],
                    agg_sh.at[pl.ds(NS * RT, N - NS * RT)])
  plsc.subcore_barrier()

  NCHUNK = SUPER // CHUNK
  bufs = (rows0, rows1)
  gsems = (sem0, sem1)
  fbufs = (rowsf, rowsf1)
  ssems = (ssem0, ssem1)
  d80s = (dst80_0, dst80_1)

  def _start(j, b):
    off = pl.multiple_of(j * CHUNK, CHUNK)
    pltpu.async_copy(xwr_hbm.at[srv.at[pl.ds(off, CHUNK)]], bufs[b], gsems[b])

  def _drain_gather(b):
    pltpu.make_async_copy(xwr_hbm.at[srv.at[pl.ds(0, CHUNK)]], bufs[b],
                          gsems[b]).wait()

  def _drain_scatter(b):
    pltpu.make_async_copy(fbufs[b], agg_sh.at[d80s[b]], ssems[b]).wait()

  def _process(j, b):
    off = pl.multiple_of(j * CHUNK, CHUNK)
    buf = bufs[b]
    fb = fbufs[b]
    d80 = d80s[b]
    for g in range(G16):
      o = pl.multiple_of(off + g * LANES, LANES)
      n16 = nv[pl.ds(o, LANES)]
      for e in range(LANES):
        nb = _bcast16(n16, e)
        row = g * LANES + e
        for c4 in range(4):
          w = buf[row, pl.ds(c4 * LANES, LANES)]
          lo = lax.bitcast_convert_type(w << 16, jnp.float32)
          hi = lax.bitcast_convert_type(w & jnp.int32(-65536), jnp.float32)
          fb[row, pl.ds(c4 * 32, LANES)] = lo * nb
          fb[row, pl.ds(c4 * 32 + LANES, LANES)] = hi * nb
    for g in range(G16):
      o = pl.multiple_of(off + g * LANES, LANES)
      d80[pl.ds(g * LANES, LANES)] = dstv[pl.ds(o, LANES)]
    pltpu.async_copy(fb, agg_sh.at[d80], ssems[b], add=True)

  def _superblk(s, _):
    ebase = wid * E_PER_TILE_B + s * SUPER
    pltpu.sync_copy(srcrel_hbm.at[pl.ds(ebase, SUPER)], srv)
    pltpu.sync_copy(dst_hbm.at[pl.ds(ebase, SUPER)], dstv)
    pltpu.sync_copy(norm_hbm.at[pl.ds(ebase, SUPER)], nv)

    _start(0, 0)
    _drain_gather(0)
    _start(1, 1)
    _process(0, 0)
    _drain_gather(1)
    _start(2, 0)
    _process(1, 1)

    def _pair(jj, _):
      j0 = pl.multiple_of(jj * 2, 2)
      _drain_gather(0)
      _start(j0 + 1, 1)
      _drain_scatter(0)
      _process(j0, 0)
      _drain_gather(1)
      _start(j0 + 2, 0)
      _drain_scatter(1)
      _process(j0 + 1, 1)
      return 0
    lax.fori_loop(1, (NCHUNK - 1) // 2, _pair, 0)
    _drain_gather(0)
    _drain_scatter(0)
    _process(NCHUNK - 1, 0)
    _drain_scatter(1)
    _drain_scatter(0)
    return 0
  lax.fori_loop(0, E_PER_TILE_B // SUPER, _superblk, 0)
  plsc.subcore_barrier()

  for j in range(RT // CHUNK):
    o = rbase + j * CHUNK
    pltpu.sync_copy(agg_sh.at[pl.ds(o, CHUNK)],
                    aggp_hbm.at[cid, pl.ds(o, CHUNK)])
  o2 = rbase + (RT // CHUNK) * CHUNK
  pltpu.sync_copy(agg_sh.at[pl.ds(o2, RT % CHUNK)],
                  aggp_hbm.at[cid, pl.ds(o2, RT % CHUNK)])

  @pl.when(sid == NS - 1)
  def _dlast():
    pltpu.sync_copy(agg_sh.at[pl.ds(NS * RT, N - NS * RT)],
                    aggp_hbm.at[cid, pl.ds(NS * RT, N - NS * RT)])


@functools.partial(
    pl.kernel,
    out_type=jax.ShapeDtypeStruct((NC, N, 128), jnp.float32),
    mesh=plsc.VectorSubcoreMesh(core_axis_name="c", subcore_axis_name="s"),
    compiler_params=pltpu.CompilerParams(use_tc_tiling_on_sc=False),
    scratch_types=(
        pltpu.VMEM_SHARED((N, 128), jnp.float32),
        pltpu.VMEM((CHUNK, 64), jnp.int32),
        pltpu.VMEM((CHUNK, 64), jnp.int32),
        pltpu.VMEM((CHUNK, 128), jnp.float32),
        pltpu.VMEM((CHUNK, 128), jnp.float32),
        pltpu.VMEM((SUPER,), jnp.int32),
        pltpu.VMEM((SUPER,), jnp.int32),
        pltpu.VMEM((SUPER,), jnp.float32),
        pltpu.VMEM((CHUNK,), jnp.int32),
        pltpu.VMEM((CHUNK,), jnp.int32),
        pltpu.SemaphoreType.DMA,
        pltpu.SemaphoreType.DMA,
        pltpu.SemaphoreType.DMA,
        pltpu.SemaphoreType.DMA,
    ),
)
def _sc_agg(xwr_hbm, srcrel_hbm, dst_hbm, norm_hbm, aggp_hbm, *rest):
  _agg_body(xwr_hbm, srcrel_hbm, dst_hbm, norm_hbm, aggp_hbm, *rest)


BLK = 1000


_TAU = np.empty((128,), np.int32)
for _w in range(64):
  _TAU[_w] = (_w // 16) * 32 + (_w % 16)
  _TAU[64 + _w] = (_w // 16) * 32 + 16 + (_w % 16)


def _xwr_body(x_ref, w_ref, o_ref):
  y = jnp.dot(x_ref[...], w_ref[0],
              preferred_element_type=jnp.float32).astype(jnp.bfloat16)
  u = lax.bitcast_convert_type(y, jnp.uint16).astype(jnp.uint32)
  packed = u[:, :64] | (u[:, 64:] << 16)
  o_ref[0] = lax.bitcast_convert_type(packed, jnp.int32)


def _xwr(x, Wr):
  Wr = Wr[:, :, _TAU]
  d = x.shape[1]
  out = pl.pallas_call(
      _xwr_body,
      grid=(R, N // BLK),
      in_specs=[
          pl.BlockSpec((BLK, d), lambda r, i: (i, 0)),
          pl.BlockSpec((1, d, 128), lambda r, i: (r, 0, 0)),
      ],
      out_specs=pl.BlockSpec((1, BLK, 64), lambda r, i: (r, i, 0)),
      out_shape=jax.ShapeDtypeStruct((R, N, 64), jnp.int32),
  )(x, Wr)
  return out.reshape(R * N, 64)


def _combine_body(aggp_ref, x_ref, wroot_ref, b_ref, o_ref, *, act):
  v = (aggp_ref[0] + aggp_ref[1]
       + jnp.dot(x_ref[...], wroot_ref[...],
                 preferred_element_type=jnp.float32)
       + b_ref[...])
  if act:
    v = jnp.where(v > 0, v, jnp.exp(v) - 1.0)
  o_ref[...] = v


def _combine(aggp, x, Wroot, b, act):
  d = x.shape[1]
  return pl.pallas_call(
      functools.partial(_combine_body, act=act),
      grid=(N // BLK,),
      in_specs=[
          pl.BlockSpec((NC, BLK, 128), lambda i: (0, i, 0)),
          pl.BlockSpec((BLK, d), lambda i: (i, 0)),
          pl.BlockSpec((d, 128), lambda i: (0, 0)),
          pl.BlockSpec((1, 128), lambda i: (0, 0)),
      ],
      out_specs=pl.BlockSpec((BLK, 128), lambda i: (i, 0)),
      out_shape=jax.ShapeDtypeStruct((N, 128), jnp.float32),
  )(aggp, x, Wroot, b.reshape(1, 128))


def _combine0_body(aggp_ref, x_ref, wroot_ref, b_ref, qe_ref, wq_ref,
                   bq_ref, batch_ref, o_ref):
  v = (aggp_ref[0] + aggp_ref[1]
       + jnp.dot(x_ref[...], wroot_ref[...],
                 preferred_element_type=jnp.float32)
       + b_ref[...])
  v = jnp.where(v > 0, v, jnp.exp(v) - 1.0)
  q = jnp.dot(qe_ref[...], wq_ref[...],
              preferred_element_type=jnp.float32) + bq_ref[...]
  q = jnp.where(q > 0, q, jnp.exp(q) - 1.0)
  onehot = (batch_ref[...] == lax.broadcasted_iota(
      jnp.int32, (1, B), 1)).astype(jnp.float32)
  qn = jnp.dot(onehot, q, preferred_element_type=jnp.float32)
  o_ref[:, :128] = v
  o_ref[:, 128:] = qn


def _combine0(aggp, x, Wroot, b, qemb, Wq, bq, batch):
  return pl.pallas_call(
      _combine0_body,
      grid=(N // BLK,),
      in_specs=[
          pl.BlockSpec((NC, BLK, 128), lambda i: (0, i, 0)),
          pl.BlockSpec((BLK, 128), lambda i: (i, 0)),
          pl.BlockSpec((128, 128), lambda i: (0, 0)),
          pl.BlockSpec((1, 128), lambda i: (0, 0)),
          pl.BlockSpec((B, 768), lambda i: (0, 0)),
          pl.BlockSpec((768, 64), lambda i: (0, 0)),
          pl.BlockSpec((1, 64), lambda i: (0, 0)),
          pl.BlockSpec((BLK, 1), lambda i: (i, 0)),
      ],
      out_specs=pl.BlockSpec((BLK, 192), lambda i: (i, 0)),
      out_shape=jax.ShapeDtypeStruct((N, 192), jnp.float32),
  )(aggp, x, Wroot, b.reshape(1, 128), qemb, Wq, bq.reshape(1, 64),
    batch.reshape(N, 1))


def kernel(x, edge_index, edge_attr, batch, question_embedding, Wq, bq,
           Wr0, Wroot0, b0, Wr1, Wroot1, b1, Wr2, Wroot2, b2,
           Wr3, Wroot3, b3):
  src = edge_index[0]
  dst = edge_index[1]
  et = edge_attr

  srcrel, norm = _sc_prep(src, dst, et)

  aggp = _sc_agg(_xwr(x, Wr0), srcrel, dst, norm)
  h = _combine0(aggp, x, Wroot0, b0, question_embedding, Wq, bq, batch)

  aggp = _sc_agg(_xwr(h, Wr1), srcrel, dst, norm)
  h = _combine(aggp, h, Wroot1, b1, act=True)

  aggp = _sc_agg(_xwr(h, Wr2), srcrel, dst, norm)
  h = _combine(aggp, h, Wroot2, b2, act=True)

  aggp = _sc_agg(_xwr(h, Wr3), srcrel, dst, norm)
  out = _combine(aggp, h, Wroot3, b3, act=False)
  return out


# speedup vs baseline: 1.0524x; 1.0524x over previous
import functools

import numpy as np

import jax
import jax.numpy as jnp
from jax import lax
from jax.experimental import pallas as pl
from jax.experimental.pallas import tpu as pltpu
from jax.experimental.pallas import tpu_sc as plsc

N = 10000
E = 320000
R = 8
B = 16

NC = 2
NS = 16
LANES = 16

CHUNK = 80
SUPER = 2000
G16 = CHUNK // LANES

E_PER_CORE = E // NC
E_PER_TILE_B = E // (NC * NS)
E_PER_TILE_P = E // NS
NR = N * R
ROWS_PER_TILE = N // NS
RT = 624


_GATHER_DNUMS = lax.GatherDimensionNumbers(
    offset_dims=(), collapsed_slice_dims=(0,), start_index_map=(0,))


def _bcast16(vec16, lane):
  idx = jnp.full((LANES, 1), lane, dtype=jnp.int32)
  return lax.gather(vec16, idx, _GATHER_DNUMS, slice_sizes=(1,),
                    mode=lax.GatherScatterMode.PROMISE_IN_BOUNDS)


def _prep_body(src_hbm, dst_hbm, et_hbm, srcrel_hbm, norm_hbm,
               cnt_sh, zbuf, onesv, dstv, etv, srcv, comp80, cv80,
               nrmv, srv, sem):
  cid = lax.axis_index("c")
  sid = lax.axis_index("s")
  wid = cid * NS + sid

  def _z(i, _):
    zbuf[pl.ds(pl.multiple_of(i * LANES, LANES), LANES)] = jnp.zeros(
        (LANES,), jnp.float32)
    return 0
  lax.fori_loop(0, SUPER // LANES, _z, 0)
  for g in range(G16):
    onesv[pl.ds(g * LANES, LANES)] = jnp.ones((LANES,), jnp.float32)

  base_z = sid * (NR // NS)
  pltpu.sync_copy(zbuf, cnt_sh.at[pl.ds(base_z, SUPER)])
  pltpu.sync_copy(zbuf, cnt_sh.at[pl.ds(base_z + SUPER, SUPER)])
  pltpu.sync_copy(zbuf.at[pl.ds(0, NR // NS - 2 * SUPER)],
                  cnt_sh.at[pl.ds(base_z + 2 * SUPER, NR // NS - 2 * SUPER)])
  plsc.subcore_barrier()

  def _count_super(s, _):
    base = sid * E_PER_TILE_P + s * SUPER
    pltpu.sync_copy(dst_hbm.at[pl.ds(base, SUPER)], dstv)
    pltpu.sync_copy(et_hbm.at[pl.ds(base, SUPER)], etv)

    def _count_chunk(j, _):
      off = pl.multiple_of(j * CHUNK, CHUNK)
      for g in range(G16):
        o = pl.multiple_of(off + g * LANES, LANES)
        comp80[pl.ds(g * LANES, LANES)] = (
            dstv[pl.ds(o, LANES)] * R + etv[pl.ds(o, LANES)])
      pltpu.sync_copy(onesv, cnt_sh.at[comp80], add=True)
      return 0
    lax.fori_loop(0, SUPER // CHUNK, _count_chunk, 0)
    return 0
  lax.fori_loop(0, E_PER_TILE_P // SUPER, _count_super, 0)
  plsc.subcore_barrier()

  def _norm_super(s, _):
    base = wid * E_PER_TILE_B + s * SUPER
    pltpu.sync_copy(dst_hbm.at[pl.ds(base, SUPER)], dstv)
    pltpu.sync_copy(et_hbm.at[pl.ds(base, SUPER)], etv)
    pltpu.sync_copy(src_hbm.at[pl.ds(base, SUPER)], srcv)

    def _srv(i, _):
      o = pl.multiple_of(i * LANES, LANES)
      srv[pl.ds(o, LANES)] = etv[pl.ds(o, LANES)] * N + srcv[pl.ds(o, LANES)]
      return 0
    lax.fori_loop(0, SUPER // LANES, _srv, 0)

    def _norm_chunk(j, _):
      off = pl.multiple_of(j * CHUNK, CHUNK)
      for g in range(G16):
        o = pl.multiple_of(off + g * LANES, LANES)
        comp80[pl.ds(g * LANES, LANES)] = (
            dstv[pl.ds(o, LANES)] * R + etv[pl.ds(o, LANES)])
      pltpu.async_copy(cnt_sh.at[comp80], cv80, sem).wait()
      for g in range(G16):
        o = pl.multiple_of(off + g * LANES, LANES)
        nrmv[pl.ds(o, LANES)] = 1.0 / jnp.maximum(
            cv80[pl.ds(g * LANES, LANES)], 1.0)
      return 0
    lax.fori_loop(0, SUPER // CHUNK, _norm_chunk, 0)

    pltpu.sync_copy(nrmv, norm_hbm.at[pl.ds(base, SUPER)])
    pltpu.sync_copy(srv, srcrel_hbm.at[pl.ds(base, SUPER)])
    return 0
  lax.fori_loop(0, E_PER_TILE_B // SUPER, _norm_super, 0)


@functools.partial(
    pl.kernel,
    out_type=(jax.ShapeDtypeStruct((E,), jnp.int32),
              jax.ShapeDtypeStruct((E,), jnp.float32)),
    mesh=plsc.VectorSubcoreMesh(core_axis_name="c", subcore_axis_name="s"),
    scratch_types=(
        pltpu.VMEM_SHARED((NR,), jnp.float32),
        pltpu.VMEM((SUPER,), jnp.float32),
        pltpu.VMEM((CHUNK,), jnp.float32),
        pltpu.VMEM((SUPER,), jnp.int32),
        pltpu.VMEM((SUPER,), jnp.int32),
        pltpu.VMEM((SUPER,), jnp.int32),
        pltpu.VMEM((CHUNK,), jnp.int32),
        pltpu.VMEM((CHUNK,), jnp.float32),
        pltpu.VMEM((SUPER,), jnp.float32),
        pltpu.VMEM((SUPER,), jnp.int32),
        pltpu.SemaphoreType.DMA,
    ),
)
def _sc_prep(src_hbm, dst_hbm, et_hbm, srcrel_hbm, norm_hbm, *rest):
  _prep_body(src_hbm, dst_hbm, et_hbm, srcrel_hbm, norm_hbm, *rest)


def _agg_body(xwr_hbm, srcrel_hbm, dst_hbm, norm_hbm, aggp_hbm,
              agg_sh, rows0, rows1, srv, dstv, nv,
              dst80_0, dst80_1, sem0, sem1, ssem0, ssem1):
  cid = lax.axis_index("c")
  sid = lax.axis_index("s")
  wid = cid * NS + sid
  rows_v = rows0

  def _z(i, _):
    for c in range(8):
      rows_v[i, pl.ds(c * LANES, LANES)] = jnp.zeros((LANES,), jnp.float32)
    return 0
  lax.fori_loop(0, CHUNK, _z, 0)
  rbase = sid * RT
  for j in range(RT // CHUNK):
    pltpu.sync_copy(rows_v, agg_sh.at[pl.ds(rbase + j * CHUNK, CHUNK)])
  pltpu.sync_copy(rows_v.at[pl.ds(0, RT % CHUNK)],
                  agg_sh.at[pl.ds(rbase + (RT // CHUNK) * CHUNK, RT % CHUNK)])

  @pl.when(sid == NS - 1)
  def _zlast():
    pltpu.sync_copy(rows_v.at[pl.ds(0, N - NS * RT)],
                    agg_sh.at[pl.ds(NS * RT, N - NS * RT)])
  plsc.subcore_barrier()

  ebase = wid * E_PER_TILE_B
  pltpu.sync_copy(srcrel_hbm.at[pl.ds(ebase, E_PER_TILE_B)], srv)
  pltpu.sync_copy(dst_hbm.at[pl.ds(ebase, E_PER_TILE_B)], dstv)
  pltpu.sync_copy(norm_hbm.at[pl.ds(ebase, E_PER_TILE_B)], nv)

  NCHUNK = E_PER_TILE_B // CHUNK
  bufs = (rows0, rows1)
  gsems = (sem0, sem1)
  d80 = dst80_0

  def _start(j, b):
    off = pl.multiple_of(j * CHUNK, CHUNK)
    pltpu.async_copy(xwr_hbm.at[srv.at[pl.ds(off, CHUNK)]], bufs[b], gsems[b])

  def _drain_gather(b):
    pltpu.make_async_copy(xwr_hbm.at[srv.at[pl.ds(0, CHUNK)]], bufs[b],
                          gsems[b]).wait()

  def _process(j, b):
    off = pl.multiple_of(j * CHUNK, CHUNK)
    buf = bufs[b]
    for g in range(G16):
      o = pl.multiple_of(off + g * LANES, LANES)
      n16 = nv[pl.ds(o, LANES)]
      for e in range(LANES):
        nb = _bcast16(n16, e)
        row = g * LANES + e
        for c in range(8):
          sl = pl.ds(c * LANES, LANES)
          buf[row, sl] = buf[row, sl] * nb
    for g in range(G16):
      o = pl.multiple_of(off + g * LANES, LANES)
      d80[pl.ds(g * LANES, LANES)] = dstv[pl.ds(o, LANES)]
    pltpu.sync_copy(buf, agg_sh.at[d80], add=True)

  _start(0, 0)

  def _pair(jj, _):
    j0 = pl.multiple_of(jj * 2, 2)
    _drain_gather(0)
    _start(j0 + 1, 1)
    _process(j0, 0)
    _drain_gather(1)
    _start(j0 + 2, 0)
    _process(j0 + 1, 1)
    return 0
  lax.fori_loop(0, (NCHUNK - 1) // 2, _pair, 0)
  _drain_gather(0)
  _process(NCHUNK - 1, 0)
  plsc.subcore_barrier()

  for j in range(RT // CHUNK):
    o = rbase + j * CHUNK
    pltpu.sync_copy(agg_sh.at[pl.ds(o, CHUNK)],
                    aggp_hbm.at[cid, pl.ds(o, CHUNK)])
  o2 = rbase + (RT // CHUNK) * CHUNK
  pltpu.sync_copy(agg_sh.at[pl.ds(o2, RT % CHUNK)],
                  aggp_hbm.at[cid, pl.ds(o2, RT % CHUNK)])

  @pl.when(sid == NS - 1)
  def _dlast():
    pltpu.sync_copy(agg_sh.at[pl.ds(NS * RT, N - NS * RT)],
                    aggp_hbm.at[cid, pl.ds(NS * RT, N - NS * RT)])


@functools.partial(
    pl.kernel,
    out_type=jax.ShapeDtypeStruct((NC, N, 128), jnp.float32),
    mesh=plsc.VectorSubcoreMesh(core_axis_name="c", subcore_axis_name="s"),
    scratch_types=(
        pltpu.VMEM_SHARED((N, 128), jnp.float32),
        pltpu.VMEM((CHUNK, 128), jnp.float32),
        pltpu.VMEM((CHUNK, 128), jnp.float32),
        pltpu.VMEM((E_PER_TILE_B,), jnp.int32),
        pltpu.VMEM((E_PER_TILE_B,), jnp.int32),
        pltpu.VMEM((E_PER_TILE_B,), jnp.float32),
        pltpu.VMEM((CHUNK,), jnp.int32),
        pltpu.VMEM((CHUNK,), jnp.int32),
        pltpu.SemaphoreType.DMA,
        pltpu.SemaphoreType.DMA,
        pltpu.SemaphoreType.DMA,
        pltpu.SemaphoreType.DMA,
    ),
)
def _sc_agg(xwr_hbm, srcrel_hbm, dst_hbm, norm_hbm, aggp_hbm, *rest):
  _agg_body(xwr_hbm, srcrel_hbm, dst_hbm, norm_hbm, aggp_hbm, *rest)


BLK = 1000


_TAU = np.empty((128,), np.int32)
for _w in range(64):
  _TAU[_w] = (_w // 16) * 32 + (_w % 16)
  _TAU[64 + _w] = (_w // 16) * 32 + 16 + (_w % 16)


def _xwr_body(x_ref, w_ref, o_ref):
  o_ref[0] = jnp.dot(x_ref[...], w_ref[0],
                     preferred_element_type=jnp.float32)


def _xwr(x, Wr):
  d = x.shape[1]
  out = pl.pallas_call(
      _xwr_body,
      grid=(R, N // BLK),
      in_specs=[
          pl.BlockSpec((BLK, d), lambda r, i: (i, 0)),
          pl.BlockSpec((1, d, 128), lambda r, i: (r, 0, 0)),
      ],
      out_specs=pl.BlockSpec((1, BLK, 128), lambda r, i: (r, i, 0)),
      out_shape=jax.ShapeDtypeStruct((R, N, 128), jnp.float32),
  )(x, Wr)
  return out.reshape(R * N, 128)


def _combine_body(aggp_ref, x_ref, wroot_ref, b_ref, o_ref, *, act):
  v = (aggp_ref[0] + aggp_ref[1]
       + jnp.dot(x_ref[...], wroot_ref[...],
                 preferred_element_type=jnp.float32)
       + b_ref[...])
  if act:
    v = jnp.where(v > 0, v, jnp.exp(v) - 1.0)
  o_ref[...] = v


def _combine(aggp, x, Wroot, b, act):
  d = x.shape[1]
  return pl.pallas_call(
      functools.partial(_combine_body, act=act),
      grid=(N // BLK,),
      in_specs=[
          pl.BlockSpec((NC, BLK, 128), lambda i: (0, i, 0)),
          pl.BlockSpec((BLK, d), lambda i: (i, 0)),
          pl.BlockSpec((d, 128), lambda i: (0, 0)),
          pl.BlockSpec((1, 128), lambda i: (0, 0)),
      ],
      out_specs=pl.BlockSpec((BLK, 128), lambda i: (i, 0)),
      out_shape=jax.ShapeDtypeStruct((N, 128), jnp.float32),
  )(aggp, x, Wroot, b.reshape(1, 128))


def _combine0_body(aggp_ref, x_ref, wroot_ref, b_ref, qe_ref, wq_ref,
                   bq_ref, batch_ref, o_ref):
  v = (aggp_ref[0] + aggp_ref[1]
       + jnp.dot(x_ref[...], wroot_ref[...],
                 preferred_element_type=jnp.float32)
       + b_ref[...])
  v = jnp.where(v > 0, v, jnp.exp(v) - 1.0)
  q = jnp.dot(qe_ref[...], wq_ref[...],
              preferred_element_type=jnp.float32) + bq_ref[...]
  q = jnp.where(q > 0, q, jnp.exp(q) - 1.0)
  onehot = (batch_ref[...] == lax.broadcasted_iota(
      jnp.int32, (1, B), 1)).astype(jnp.float32)
  qn = jnp.dot(onehot, q, preferred_element_type=jnp.float32)
  o_ref[:, :128] = v
  o_ref[:, 128:] = qn


def _combine0(aggp, x, Wroot, b, qemb, Wq, bq, batch):
  return pl.pallas_call(
      _combine0_body,
      grid=(N // BLK,),
      in_specs=[
          pl.BlockSpec((NC, BLK, 128), lambda i: (0, i, 0)),
          pl.BlockSpec((BLK, 128), lambda i: (i, 0)),
          pl.BlockSpec((128, 128), lambda i: (0, 0)),
          pl.BlockSpec((1, 128), lambda i: (0, 0)),
          pl.BlockSpec((B, 768), lambda i: (0, 0)),
          pl.BlockSpec((768, 64), lambda i: (0, 0)),
          pl.BlockSpec((1, 64), lambda i: (0, 0)),
          pl.BlockSpec((BLK, 1), lambda i: (i, 0)),
      ],
      out_specs=pl.BlockSpec((BLK, 192), lambda i: (i, 0)),
      out_shape=jax.ShapeDtypeStruct((N, 192), jnp.float32),
  )(aggp, x, Wroot, b.reshape(1, 128), qemb, Wq, bq.reshape(1, 64),
    batch.reshape(N, 1))


def kernel(x, edge_index, edge_attr, batch, question_embedding, Wq, bq,
           Wr0, Wroot0, b0, Wr1, Wroot1, b1, Wr2, Wroot2, b2,
           Wr3, Wroot3, b3):
  src = edge_index[0]
  dst = edge_index[1]
  et = edge_attr

  srcrel, norm = _sc_prep(src, dst, et)

  aggp = _sc_agg(_xwr(x, Wr0), srcrel, dst, norm)
  h = _combine0(aggp, x, Wroot0, b0, question_embedding, Wq, bq, batch)

  aggp = _sc_agg(_xwr(h, Wr1), srcrel, dst, norm)
  h = _combine(aggp, h, Wroot1, b1, act=True)

  aggp = _sc_agg(_xwr(h, Wr2), srcrel, dst, norm)
  h = _combine(aggp, h, Wroot2, b2, act=True)

  aggp = _sc_agg(_xwr(h, Wr3), srcrel, dst, norm)
  out = _combine(aggp, h, Wroot3, b3, act=False)
  return out
